# baseline (speedup 1.0000x reference)
import functools
import jax
import jax.numpy as jnp
from jax.experimental import pallas as pl
from jax.experimental.pallas import tpu as pltpu

N, L, K = 320000, 128, 16
BLK = 32000
NBLK = N // BLK
DENOM = float(L * K * (K - 1))


def _body(tt_ref, phi_ref, out_ref, acc_ref, cnt_ref):
    i = pl.program_id(0)

    @pl.when(i == 0)
    def _init():
        acc_ref[...] = jnp.zeros_like(acc_ref)
        cnt_ref[...] = jnp.zeros_like(cnt_ref)

    mt = tt_ref[...].astype(jnp.float32)
    cnt_ref[...] += jnp.sum(mt, axis=1, keepdims=True)
    acc_ref[...] += jax.lax.dot_general(
        mt, phi_ref[...], (((1,), (0,)), ((), ())),
        preferred_element_type=jnp.float32)

    @pl.when(i == NBLK - 1)
    def _epilogue():
        s = acc_ref[...]
        c_col = cnt_ref[...]
        rows = jax.lax.broadcasted_iota(jnp.int32, (K, K), 0)
        cols = jax.lax.broadcasted_iota(jnp.int32, (K, K), 1)
        eye = (rows == cols).astype(jnp.float32)
        c_row = jax.lax.dot_general(
            c_col, eye, (((0,), (0,)), ((), ())),
            preferred_element_type=jnp.float32)
        gram_s = jax.lax.dot_general(
            s, s, (((1,), (1,)), ((), ())),
            preferred_element_type=jnp.float32)
        gram = gram_s / (c_col * c_row)
        sq_col = jnp.sum(gram * eye, axis=1, keepdims=True)
        sq_row = jnp.sum(gram * eye, axis=0, keepdims=True)
        d2 = sq_col + sq_row - 2.0 * gram
        dist = jnp.sqrt(jnp.maximum(d2, 0.0))
        offdiag = (rows != cols).astype(jnp.float32)
        out_ref[0, 0] = jnp.sum(dist * offdiag) / DENOM


@jax.jit
def kernel(phi_x, t):
    tt = jnp.transpose(t)
    out = pl.pallas_call(
        _body,
        grid=(NBLK,),
        in_specs=[
            pl.BlockSpec((K, BLK), lambda i: (0, i)),
            pl.BlockSpec((BLK, L), lambda i: (i, 0)),
        ],
        out_specs=pl.BlockSpec(memory_space=pltpu.SMEM),
        out_shape=jax.ShapeDtypeStruct((1, 1), jnp.float32),
        scratch_shapes=[
            pltpu.VMEM((K, L), jnp.float32),
            pltpu.VMEM((K, 1), jnp.float32),
        ],
    )(tt, phi_x)
    return out[0, 0]


# speedup vs baseline: 1.0623x; 1.0623x over previous
import functools
import jax
import jax.numpy as jnp
from jax.experimental import pallas as pl
from jax.experimental.pallas import tpu as pltpu

N, L, K = 320000, 128, 16
BLK = 12800
NBLK = N // BLK
DENOM = float(L * K * (K - 1))


def _body(tt_ref, phi_ref, out_ref, acc_ref, cnt_ref):
    i = pl.program_id(0)

    @pl.when(i == 0)
    def _init():
        acc_ref[...] = jnp.zeros_like(acc_ref)
        cnt_ref[...] = jnp.zeros_like(cnt_ref)

    mt = tt_ref[...].astype(jnp.float32)
    cnt_ref[...] += jnp.sum(mt, axis=1, keepdims=True)
    acc_ref[...] += jax.lax.dot_general(
        mt, phi_ref[...], (((1,), (0,)), ((), ())),
        preferred_element_type=jnp.float32)

    @pl.when(i == NBLK - 1)
    def _epilogue():
        s = acc_ref[...]
        c_col = cnt_ref[...]
        rows = jax.lax.broadcasted_iota(jnp.int32, (K, K), 0)
        cols = jax.lax.broadcasted_iota(jnp.int32, (K, K), 1)
        eye = (rows == cols).astype(jnp.float32)
        c_row = jax.lax.dot_general(
            c_col, eye, (((0,), (0,)), ((), ())),
            preferred_element_type=jnp.float32)
        gram_s = jax.lax.dot_general(
            s, s, (((1,), (1,)), ((), ())),
            preferred_element_type=jnp.float32)
        gram = gram_s / (c_col * c_row)
        sq_col = jnp.sum(gram * eye, axis=1, keepdims=True)
        sq_row = jnp.sum(gram * eye, axis=0, keepdims=True)
        d2 = sq_col + sq_row - 2.0 * gram
        dist = jnp.sqrt(jnp.maximum(d2, 0.0))
        offdiag = (rows != cols).astype(jnp.float32)
        out_ref[0, 0] = jnp.sum(dist * offdiag) / DENOM


@jax.jit
def kernel(phi_x, t):
    tt = jnp.transpose(t)
    out = pl.pallas_call(
        _body,
        grid=(NBLK,),
        in_specs=[
            pl.BlockSpec((K, BLK), lambda i: (0, i)),
            pl.BlockSpec((BLK, L), lambda i: (i, 0)),
        ],
        out_specs=pl.BlockSpec(memory_space=pltpu.SMEM),
        out_shape=jax.ShapeDtypeStruct((1, 1), jnp.float32),
        scratch_shapes=[
            pltpu.VMEM((K, L), jnp.float32),
            pltpu.VMEM((K, 1), jnp.float32),
        ],
    )(tt, phi_x)
    return out[0, 0]


# speedup vs baseline: 1.0639x; 1.0015x over previous
import jax
import jax.numpy as jnp
from jax.experimental import pallas as pl
from jax.experimental.pallas import tpu as pltpu

N, L, K = 320000, 128, 16
BLK = 12800
NBLK = N // BLK
DENOM = float(L * K * (K - 1))


def _body(tt_ref, phi_ref, out_ref, acc_ref, cnt_ref):
    i = pl.program_id(0)

    @pl.when(i == 0)
    def _init():
        acc_ref[...] = jnp.zeros_like(acc_ref)
        cnt_ref[...] = jnp.zeros_like(cnt_ref)

    mt = tt_ref[...].astype(jnp.float32)
    cnt_ref[...] += jnp.sum(mt, axis=1, keepdims=True)
    acc_ref[...] += jax.lax.dot_general(
        mt, phi_ref[...], (((1,), (0,)), ((), ())),
        preferred_element_type=jnp.float32)

    @pl.when(i == NBLK - 1)
    def _epilogue():
        s = acc_ref[...]
        c_col = cnt_ref[...]
        rows = jax.lax.broadcasted_iota(jnp.int32, (K, K), 0)
        cols = jax.lax.broadcasted_iota(jnp.int32, (K, K), 1)
        eye = (rows == cols).astype(jnp.float32)
        c_row = jax.lax.dot_general(
            c_col, eye, (((0,), (0,)), ((), ())),
            preferred_element_type=jnp.float32)
        gram_s = jax.lax.dot_general(
            s, s, (((1,), (1,)), ((), ())),
            preferred_element_type=jnp.float32)
        gram = gram_s / (c_col * c_row)
        sq_col = jnp.sum(gram * eye, axis=1, keepdims=True)
        sq_row = jnp.sum(gram * eye, axis=0, keepdims=True)
        d2 = sq_col + sq_row - 2.0 * gram
        dist = jnp.sqrt(jnp.maximum(d2, 0.0))
        offdiag = (rows != cols).astype(jnp.float32)
        out_ref[0, 0] = jnp.sum(dist * offdiag) / DENOM


@jax.jit
def kernel(phi_x, t):
    tt = jnp.transpose(t)
    out = pl.pallas_call(
        _body,
        grid=(NBLK,),
        in_specs=[
            pl.BlockSpec((K, BLK), lambda i: (0, i)),
            pl.BlockSpec((BLK, L), lambda i: (i, 0)),
        ],
        out_specs=pl.BlockSpec(memory_space=pltpu.SMEM),
        out_shape=jax.ShapeDtypeStruct((1, 1), jnp.float32),
        scratch_shapes=[
            pltpu.VMEM((K, L), jnp.float32),
            pltpu.VMEM((K, 1), jnp.float32),
        ],
    )(tt, phi_x)
    return out[0, 0]
